# Initial kernel scaffold; baseline (speedup 1.0000x reference)
import jax
import jax.numpy as jnp
from jax.experimental import pallas as pl


def kernel(X, tlut):
    raise NotImplementedError("write your pallas kernel here")


# speedup vs baseline: 3.0442x; 3.0442x over previous
import jax
import jax.numpy as jnp
from jax import lax
from jax.experimental import pallas as pl
from jax.experimental.pallas import tpu as pltpu

B = 262144
K = 256
V = 4
BM = 2048


def _vq_body(x_ref, tlutT_ref, tlut_ref, hat_ref, state_ref):
    x = x_ref[...]
    tT = tlutT_ref[...]
    tl = tlut_ref[...]
    x2 = jnp.sum(x * x, axis=1, keepdims=True)
    t2 = jnp.sum(tT * tT, axis=0, keepdims=True)
    xt = lax.dot_general(x, tT, (((1,), (0,)), ((), ())),
                         preferred_element_type=jnp.float32)
    d2 = jnp.maximum(x2 - 2.0 * xt + t2, 0.0)
    m = jnp.min(d2, axis=1, keepdims=True)
    lanes = lax.broadcasted_iota(jnp.int32, (BM, K), 1)
    idx = jnp.min(jnp.where(d2 == m, lanes, K), axis=1, keepdims=True)
    onehot = (lanes == idx).astype(jnp.float32)
    hat = lax.dot_general(onehot, tl, (((1,), (0,)), ((), ())),
                          preferred_element_type=jnp.float32)
    hat_ref[...] = hat
    state_ref[...] = idx


def kernel(X, tlut):
    tlutT = tlut.T
    hat, state = pl.pallas_call(
        _vq_body,
        grid=(B // BM,),
        in_specs=[
            pl.BlockSpec((BM, V), lambda i: (i, 0)),
            pl.BlockSpec((V, K), lambda i: (0, 0)),
            pl.BlockSpec((K, V), lambda i: (0, 0)),
        ],
        out_specs=[
            pl.BlockSpec((BM, V), lambda i: (i, 0)),
            pl.BlockSpec((BM, 1), lambda i: (i, 0)),
        ],
        out_shape=[
            jax.ShapeDtypeStruct((B, V), jnp.float32),
            jax.ShapeDtypeStruct((B, 1), jnp.int32),
        ],
    )(X, tlutT, tlut)
    return hat, state.reshape(B)


# speedup vs baseline: 3.5985x; 1.1821x over previous
import jax
import jax.numpy as jnp
from jax import lax
from jax.experimental import pallas as pl
from jax.experimental.pallas import tpu as pltpu

B = 262144
K = 256
V = 4
BM = 4096


def _vq_body(x_ref, tlutT_ref, tlut_ref, hat_ref, state_ref):
    x = x_ref[...]
    tT = tlutT_ref[...]
    tl = tlut_ref[...]
    t2 = jnp.sum(tT * tT, axis=0, keepdims=True)
    xt = lax.dot_general(x, tT, (((1,), (0,)), ((), ())),
                         preferred_element_type=jnp.float32)
    d2 = t2 - 2.0 * xt
    m = jnp.min(d2, axis=1, keepdims=True)
    lanes = lax.broadcasted_iota(jnp.int32, (BM, K), 1)
    idx = jnp.min(jnp.where(d2 == m, lanes, K), axis=1, keepdims=True)
    onehot = (lanes == idx).astype(jnp.float32)
    hat = lax.dot_general(onehot, tl, (((1,), (0,)), ((), ())),
                          preferred_element_type=jnp.float32)
    hat_ref[...] = hat
    state_ref[...] = idx


def kernel(X, tlut):
    tlutT = tlut.T
    hat, state = pl.pallas_call(
        _vq_body,
        grid=(B // BM,),
        in_specs=[
            pl.BlockSpec((BM, V), lambda i: (i, 0)),
            pl.BlockSpec((V, K), lambda i: (0, 0)),
            pl.BlockSpec((K, V), lambda i: (0, 0)),
        ],
        out_specs=[
            pl.BlockSpec((BM, V), lambda i: (i, 0)),
            pl.BlockSpec((BM, 1), lambda i: (i, 0)),
        ],
        out_shape=[
            jax.ShapeDtypeStruct((B, V), jnp.float32),
            jax.ShapeDtypeStruct((B, 1), jnp.int32),
        ],
    )(X, tlutT, tlut)
    return hat, state.reshape(B)
